# baseline (speedup 1.0000x reference)
import functools

import jax
import jax.numpy as jnp
import numpy as np
from jax import lax
from jax.experimental import pallas as pl
from jax.experimental.pallas import tpu as pltpu
from jax.experimental.pallas import tpu_sc as plsc

N1 = 256
B = 1024
SIZES = tuple(range(5, 13))
NS_OUT = len(SIZES)
OUT_ROW = 12 * 12 * NS_OUT
LANES = 16
NVEC = OUT_ROW // LANES

_scale_idx = {s: [i * (N1 - 1) // (s - 1) for i in range(s)] for s in SIZES}
_union = sorted({i for v in _scale_idx.values() for i in v})
NU = len(_union)
NU_PAD = 48
ZROW = NU_PAD
_pos = {v: i for i, v in enumerate(_union)}

_u48 = np.array(_union + [0] * (NU_PAD - NU), dtype=np.int32)

_ridx = np.zeros(OUT_ROW, dtype=np.int32)
_cidx = np.zeros(OUT_ROW, dtype=np.int32)
for r in range(12):
    for c in range(12):
        for si, s in enumerate(SIZES):
            k = (r * 12 + c) * NS_OUT + si
            if r < s and c < s:
                _ridx[k] = _pos[_scale_idx[s][r]]
                _cidx[k] = _scale_idx[s][c]
            else:
                _ridx[k] = ZROW
                _cidx[k] = 0

NC, NSUB = 2, 16
NW = NC * NSUB
B_PER_W = B // NW


def _sc_body(in_hbm, u48_hbm, ridx_hbm, cidx_hbm, out_hbm,
             u_v, ridx_v, cidx_v, bidx_v, rows_v, out_v, sem):
    wid = lax.axis_index("s") * NC + lax.axis_index("c")

    pltpu.sync_copy(u48_hbm, u_v)
    pltpu.sync_copy(ridx_hbm, ridx_v)
    pltpu.sync_copy(cidx_hbm, cidx_v)

    zeros = jnp.zeros((LANES,), jnp.float32)
    for j in range(N1 // LANES):
        rows_v[ZROW, pl.ds(j * LANES, LANES)] = zeros

    def per_batch(bl, _):
        b = wid * B_PER_W + bl
        for j in range(NU_PAD // LANES):
            sl = pl.ds(j * LANES, LANES)
            bidx_v[sl] = u_v[sl] + b * N1
        pltpu.async_copy(in_hbm.at[bidx_v], rows_v.at[pl.ds(0, NU_PAD)],
                         sem).wait()

        def gather_vec(k, _):
            sl = pl.ds(k * LANES, LANES)
            out_v[sl] = plsc.load_gather(rows_v, [ridx_v[sl], cidx_v[sl]])
            return 0

        lax.fori_loop(0, NVEC, gather_vec, 0, unroll=8)
        pltpu.sync_copy(out_v, out_hbm.at[b])
        return 0

    lax.fori_loop(0, B_PER_W, per_batch, 0)


@jax.jit
def kernel(inputs):
    in2d = inputs.reshape(B * N1, N1)
    mesh = plsc.VectorSubcoreMesh(core_axis_name="c", subcore_axis_name="s")
    run = pl.kernel(
        _sc_body,
        out_type=jax.ShapeDtypeStruct((B, OUT_ROW), jnp.float32),
        mesh=mesh,
        compiler_params=pltpu.CompilerParams(use_tc_tiling_on_sc=False,
                                             needs_layout_passes=False),
        scratch_types=[
            pltpu.VMEM((NU_PAD,), jnp.int32),
            pltpu.VMEM((OUT_ROW,), jnp.int32),
            pltpu.VMEM((OUT_ROW,), jnp.int32),
            pltpu.VMEM((NU_PAD,), jnp.int32),
            pltpu.VMEM((NU_PAD + 1, N1), jnp.float32),
            pltpu.VMEM((OUT_ROW,), jnp.float32),
            pltpu.SemaphoreType.DMA,
        ],
    )
    out = run(in2d, jnp.asarray(_u48), jnp.asarray(_ridx), jnp.asarray(_cidx))
    return out.reshape(B, 12, 12, NS_OUT)


# speedup vs baseline: 1.1429x; 1.1429x over previous
import jax
import jax.numpy as jnp
import numpy as np
from jax import lax
from jax.experimental import pallas as pl
from jax.experimental.pallas import tpu as pltpu
from jax.experimental.pallas import tpu_sc as plsc

N1 = 256
B = 1024
SIZES = tuple(range(5, 13))
NS_OUT = len(SIZES)
OUT_ROW = 12 * 12 * NS_OUT
LANES = 16
NVEC = OUT_ROW // LANES

_scale_idx = {s: [i * (N1 - 1) // (s - 1) for i in range(s)] for s in SIZES}
_union = sorted({i for v in _scale_idx.values() for i in v})
NU = len(_union)
NU_PAD = 48
ZROW = NU_PAD
_pos = {v: i for i, v in enumerate(_union)}

_u48 = np.array(_union + [0] * (NU_PAD - NU), dtype=np.int32)

_ridx = np.zeros(OUT_ROW, dtype=np.int32)
_cidx = np.zeros(OUT_ROW, dtype=np.int32)
for r in range(12):
    for c in range(12):
        for si, s in enumerate(SIZES):
            k = (r * 12 + c) * NS_OUT + si
            if r < s and c < s:
                _ridx[k] = _pos[_scale_idx[s][r]]
                _cidx[k] = _scale_idx[s][c]
            else:
                _ridx[k] = ZROW
                _cidx[k] = 0

NC, NSUB = 2, 16
NW = NC * NSUB
B_PER_W = B // NW
NBUF = 8
NGRP = B_PER_W // NBUF


def _sc_body(in_hbm, u48_hbm, ridx_hbm, cidx_hbm, out_hbm,
             u_v, ridx_v, cidx_v, bidx_v, rows_v, out_v, semg, semo):
    wid = lax.axis_index("s") * NC + lax.axis_index("c")
    base = wid * B_PER_W

    pltpu.sync_copy(u48_hbm, u_v)
    pltpu.sync_copy(ridx_hbm, ridx_v)
    pltpu.sync_copy(cidx_hbm, cidx_v)

    zeros = jnp.zeros((LANES,), jnp.float32)
    for i in range(NBUF):
        for j in range(N1 // LANES):
            rows_v[i, ZROW, pl.ds(j * LANES, LANES)] = zeros

    def fill_bidx(i, b):
        for j in range(NU_PAD // LANES):
            sl = pl.ds(j * LANES, LANES)
            bidx_v[i, sl] = u_v[sl] + b * N1

    def fire_gather(i):
        return pltpu.async_copy(in_hbm.at[bidx_v.at[i]],
                                rows_v.at[i, pl.ds(0, NU_PAD)], semg.at[i])

    for i in range(NBUF):
        fill_bidx(i, base + i)
        fire_gather(i)

    def per_group(g, _):
        for i in range(NBUF):
            b = base + g * NBUF + i
            pltpu.make_async_copy(in_hbm.at[bidx_v.at[i]],
                                  rows_v.at[i, pl.ds(0, NU_PAD)],
                                  semg.at[i]).wait()

            @pl.when(g > 0)
            def _():
                pltpu.make_async_copy(out_hbm.at[0], out_v.at[i],
                                      semo.at[i]).wait()

            def gather_vec(k, _):
                sl = pl.ds(k * LANES, LANES)
                out_v[i, sl] = plsc.load_gather(
                    rows_v.at[i], [ridx_v[sl], cidx_v[sl]])
                return 0

            lax.fori_loop(0, NVEC, gather_vec, 0, unroll=8)
            pltpu.async_copy(out_v.at[i], out_hbm.at[b], semo.at[i])

            @pl.when(g < NGRP - 1)
            def _():
                fill_bidx(i, b + NBUF)
                fire_gather(i)
        return 0

    lax.fori_loop(0, NGRP, per_group, 0)

    for i in range(NBUF):
        pltpu.make_async_copy(out_hbm.at[0], out_v.at[i], semo.at[i]).wait()


@jax.jit
def kernel(inputs):
    in2d = inputs.reshape(B * N1, N1)
    mesh = plsc.VectorSubcoreMesh(core_axis_name="c", subcore_axis_name="s")
    run = pl.kernel(
        _sc_body,
        out_type=jax.ShapeDtypeStruct((B, OUT_ROW), jnp.float32),
        mesh=mesh,
        compiler_params=pltpu.CompilerParams(use_tc_tiling_on_sc=False,
                                             needs_layout_passes=False),
        scratch_types=[
            pltpu.VMEM((NU_PAD,), jnp.int32),
            pltpu.VMEM((OUT_ROW,), jnp.int32),
            pltpu.VMEM((OUT_ROW,), jnp.int32),
            pltpu.VMEM((NBUF, NU_PAD), jnp.int32),
            pltpu.VMEM((NBUF, NU_PAD + 1, N1), jnp.float32),
            pltpu.VMEM((NBUF, OUT_ROW), jnp.float32),
            pltpu.SemaphoreType.DMA((NBUF,)),
            pltpu.SemaphoreType.DMA((NBUF,)),
        ],
    )
    out = run(in2d, jnp.asarray(_u48), jnp.asarray(_ridx), jnp.asarray(_cidx))
    return out.reshape(B, 12, 12, NS_OUT)


# speedup vs baseline: 1.1437x; 1.0007x over previous
import jax
import jax.numpy as jnp
import numpy as np
from jax import lax
from jax.experimental import pallas as pl
from jax.experimental.pallas import tpu as pltpu
from jax.experimental.pallas import tpu_sc as plsc

N1 = 256
B = 1024
SIZES = tuple(range(5, 13))
NS_OUT = len(SIZES)
OUT_ROW = 12 * 12 * NS_OUT
LANES = 16
NVEC = OUT_ROW // LANES

_scale_idx = {s: [i * (N1 - 1) // (s - 1) for i in range(s)] for s in SIZES}
_union = sorted({i for v in _scale_idx.values() for i in v})
NU = len(_union)
NU_PAD = 48
ZROW = NU_PAD
_pos = {v: i for i, v in enumerate(_union)}

_u48 = np.array(_union + [0] * (NU_PAD - NU), dtype=np.int32)

_ridx = np.zeros(OUT_ROW, dtype=np.int32)
_cidx = np.zeros(OUT_ROW, dtype=np.int32)
for r in range(12):
    for c in range(12):
        for si, s in enumerate(SIZES):
            k = (r * 12 + c) * NS_OUT + si
            if r < s and c < s:
                _ridx[k] = _pos[_scale_idx[s][r]]
                _cidx[k] = _scale_idx[s][c]
            else:
                _ridx[k] = ZROW
                _cidx[k] = 0

NC, NSUB = 2, 16
NW = NC * NSUB
B_PER_W = B // NW
NBUF = 8
NGRP = B_PER_W // NBUF


def _sc_body(in_hbm, u48_hbm, ridx_hbm, cidx_hbm, out_hbm,
             u_v, ridx_v, cidx_v, rows_v, out_v, semg, semo):
    wid = lax.axis_index("s") * NC + lax.axis_index("c")
    base = wid * B_PER_W

    pltpu.sync_copy(u48_hbm, u_v)
    pltpu.sync_copy(ridx_hbm, ridx_v)
    pltpu.sync_copy(cidx_hbm, cidx_v)

    zeros = jnp.zeros((LANES,), jnp.float32)
    for i in range(NBUF):
        for j in range(N1 // LANES):
            rows_v[i, ZROW, pl.ds(j * LANES, LANES)] = zeros

    def gather_cp(i, b):
        return pltpu.make_async_copy(in_hbm.at[b].at[u_v],
                                     rows_v.at[i, pl.ds(0, NU_PAD)],
                                     semg.at[i])

    for i in range(NBUF):
        gather_cp(i, base + i).start()

    def per_group(g, _):
        for i in range(NBUF):
            b = base + g * NBUF + i
            gather_cp(i, b).wait()

            @pl.when(g > 0)
            def _():
                pltpu.make_async_copy(out_hbm.at[0], out_v.at[i],
                                      semo.at[i]).wait()

            def gather_vec(k, _):
                sl = pl.ds(k * LANES, LANES)
                out_v[i, sl] = plsc.load_gather(
                    rows_v.at[i], [ridx_v[sl], cidx_v[sl]])
                return 0

            lax.fori_loop(0, NVEC, gather_vec, 0, unroll=8)
            pltpu.async_copy(out_v.at[i], out_hbm.at[b], semo.at[i])

            @pl.when(g < NGRP - 1)
            def _():
                gather_cp(i, b + NBUF).start()
        return 0

    lax.fori_loop(0, NGRP, per_group, 0)

    for i in range(NBUF):
        pltpu.make_async_copy(out_hbm.at[0], out_v.at[i], semo.at[i]).wait()


@jax.jit
def kernel(inputs):
    mesh = plsc.VectorSubcoreMesh(core_axis_name="c", subcore_axis_name="s")
    run = pl.kernel(
        _sc_body,
        out_type=jax.ShapeDtypeStruct((B, OUT_ROW), jnp.float32),
        mesh=mesh,
        compiler_params=pltpu.CompilerParams(use_tc_tiling_on_sc=False,
                                             needs_layout_passes=False),
        scratch_types=[
            pltpu.VMEM((NU_PAD,), jnp.int32),
            pltpu.VMEM((OUT_ROW,), jnp.int32),
            pltpu.VMEM((OUT_ROW,), jnp.int32),
            pltpu.VMEM((NBUF, NU_PAD + 1, N1), jnp.float32),
            pltpu.VMEM((NBUF, OUT_ROW), jnp.float32),
            pltpu.SemaphoreType.DMA((NBUF,)),
            pltpu.SemaphoreType.DMA((NBUF,)),
        ],
    )
    out = run(inputs, jnp.asarray(_u48), jnp.asarray(_ridx), jnp.asarray(_cidx))
    return out.reshape(B, 12, 12, NS_OUT)


# speedup vs baseline: 4.1399x; 3.6198x over previous
import jax
import jax.numpy as jnp
import numpy as np
from jax import lax
from jax.experimental import pallas as pl
from jax.experimental.pallas import tpu as pltpu
from jax.experimental.pallas import tpu_sc as plsc

N1 = 256
B = 1024
SIZES = tuple(range(5, 13))
NS_OUT = len(SIZES)
OUT_ROW = 12 * 12 * NS_OUT
LANES = 16
NVEC = OUT_ROW // LANES

_scale_idx = {s: [i * (N1 - 1) // (s - 1) for i in range(s)] for s in SIZES}
_union = sorted({i for v in _scale_idx.values() for i in v})
NU = len(_union)
NU_PAD = 48
ZROW = NU_PAD
_pos = {v: i for i, v in enumerate(_union)}

_u48 = np.array(_union + [0] * (NU_PAD - NU), dtype=np.int32)

_ridx = np.zeros(OUT_ROW, dtype=np.int32)
_cidx = np.zeros(OUT_ROW, dtype=np.int32)
for r in range(12):
    for c in range(12):
        for si, s in enumerate(SIZES):
            k = (r * 12 + c) * NS_OUT + si
            if r < s and c < s:
                _ridx[k] = _pos[_scale_idx[s][r]]
                _cidx[k] = _scale_idx[s][c]
            else:
                _ridx[k] = ZROW
                _cidx[k] = 0

NC, NSUB = 2, 16
NW = NC * NSUB
B_PER_W = B // NW
NBUF = 8
NGRP = B_PER_W // NBUF


def _sc_body(in_hbm, u48_hbm, ridx_hbm, cidx_hbm, out_hbm,
             u_v, ridx_v, cidx_v, rows_v, out_v, semg, semo):
    wid = lax.axis_index("s") * NC + lax.axis_index("c")
    base = wid * B_PER_W

    pltpu.sync_copy(u48_hbm, u_v)
    pltpu.sync_copy(ridx_hbm, ridx_v)
    pltpu.sync_copy(cidx_hbm, cidx_v)

    zeros = jnp.zeros((LANES,), jnp.float32)
    for i in range(NBUF):
        for j in range(N1 // LANES):
            rows_v[i, ZROW, pl.ds(j * LANES, LANES)] = zeros

    def gather_cp(i, b):
        return pltpu.make_async_copy(in_hbm.at[b].at[u_v],
                                     rows_v.at[i, pl.ds(0, NU_PAD)],
                                     semg.at[i])

    for i in range(NBUF):
        gather_cp(i, base + i).start()

    def per_group(g, _):
        for i in range(NBUF):
            b = base + g * NBUF + i
            gather_cp(i, b).wait()

            @pl.when(g > 0)
            def _():
                pltpu.make_async_copy(out_hbm.at[0], out_v.at[i],
                                      semo.at[i]).wait()

            def gather_vec(k, _):
                sl = pl.ds(k * LANES, LANES)
                out_v[i, sl] = plsc.load_gather(
                    rows_v.at[i], [ridx_v[sl], cidx_v[sl]])
                return 0

            lax.fori_loop(0, NVEC, gather_vec, 0, unroll=8)
            pltpu.async_copy(out_v.at[i], out_hbm.at[b], semo.at[i])

            @pl.when(g < NGRP - 1)
            def _():
                gather_cp(i, b + NBUF).start()
        return 0

    lax.fori_loop(0, NGRP, per_group, 0)

    for i in range(NBUF):
        pltpu.make_async_copy(out_hbm.at[0], out_v.at[i], semo.at[i]).wait()


@jax.jit
def kernel(inputs):
    mesh = plsc.VectorSubcoreMesh(core_axis_name="c", subcore_axis_name="s")
    run = pl.kernel(
        _sc_body,
        out_type=jax.ShapeDtypeStruct((B, OUT_ROW), jnp.float32),
        mesh=mesh,
        compiler_params=pltpu.CompilerParams(use_tc_tiling_on_sc=True,
                                             needs_layout_passes=False),
        scratch_types=[
            pltpu.VMEM((NU_PAD,), jnp.int32),
            pltpu.VMEM((OUT_ROW,), jnp.int32),
            pltpu.VMEM((OUT_ROW,), jnp.int32),
            pltpu.VMEM((NBUF, NU_PAD + 1, N1), jnp.float32),
            pltpu.VMEM((NBUF, OUT_ROW), jnp.float32),
            pltpu.SemaphoreType.DMA((NBUF,)),
            pltpu.SemaphoreType.DMA((NBUF,)),
        ],
    )
    out = run(inputs, jnp.asarray(_u48), jnp.asarray(_ridx), jnp.asarray(_cidx))
    return out.reshape(B, 12, 12, NS_OUT)


# speedup vs baseline: 4.8324x; 1.1673x over previous
import jax
import jax.numpy as jnp
import numpy as np
from jax import lax
from jax.experimental import pallas as pl
from jax.experimental.pallas import tpu as pltpu
from jax.experimental.pallas import tpu_sc as plsc

N1 = 256
B = 1024
SIZES = tuple(range(5, 13))
NS_OUT = len(SIZES)
OUT_ROW = 12 * 12 * NS_OUT
LANES = 16

_scale_idx = {s: [i * (N1 - 1) // (s - 1) for i in range(s)] for s in SIZES}
_union = sorted({i for v in _scale_idx.values() for i in v})
NU = len(_union)
_pos = {v: i for i, v in enumerate(_union)}
NU_PAD = 48
_u43 = np.array(_union + [0] * (NU_PAD - NU), dtype=np.int32)

_gr, _gc, _sp = [], [], []
for r in range(12):
    for c in range(12):
        for si, s in enumerate(SIZES):
            if r < s and c < s:
                _gr.append(_pos[_scale_idx[s][r]])
                _gc.append(_scale_idx[s][c])
                _sp.append((r * 12 + c) * NS_OUT + si)
NVALID = len(_gr)
NVEC = -(-NVALID // LANES)
NTAIL = NVALID - (NVEC - 1) * LANES
_pad = NVEC * LANES - NVALID
_gr = np.array(_gr + [0] * _pad, dtype=np.int32)
_gc = np.array(_gc + [0] * _pad, dtype=np.int32)
_sp = np.array(_sp + [0] * _pad, dtype=np.int32)

NC, NSUB = 2, 16
NW = NC * NSUB
B_PER_W = B // NW
NBUF = 8
NGRP = B_PER_W // NBUF


def _sc_body(in_hbm, u43_hbm, gr_hbm, gc_hbm, sp_hbm, out_hbm,
             u_v, gr_v, gc_v, sp_v, rows_v, *rest):
    out_vs = rest[:NBUF]
    semg, semo = rest[NBUF], rest[NBUF + 1]
    wid = lax.axis_index("s") * NC + lax.axis_index("c")
    base = wid * B_PER_W

    pltpu.sync_copy(u43_hbm, u_v)
    pltpu.sync_copy(gr_hbm, gr_v)
    pltpu.sync_copy(gc_hbm, gc_v)
    pltpu.sync_copy(sp_hbm, sp_v)

    zeros = jnp.zeros((LANES,), jnp.float32)
    for i in range(NBUF):
        for j in range(OUT_ROW // LANES):
            out_vs[i][pl.ds(j * LANES, LANES)] = zeros

    def gather_cp(i, b):
        return pltpu.make_async_copy(in_hbm.at[b].at[u_v],
                                     rows_v.at[i], semg.at[i])

    for i in range(NBUF):
        gather_cp(i, base + i).start()

    tail_mask = lax.iota(jnp.int32, LANES) < NTAIL

    def per_group(g, _):
        for i in range(NBUF):
            b = base + g * NBUF + i
            gather_cp(i, b).wait()

            @pl.when(g > 0)
            def _():
                pltpu.make_async_copy(out_hbm.at[0], out_vs[i],
                                      semo.at[i]).wait()

            def move_vec(k, _):
                sl = pl.ds(k * LANES, LANES)
                vals = plsc.load_gather(rows_v.at[i], [gr_v[sl], gc_v[sl]])
                plsc.store_scatter(out_vs[i], [sp_v[sl]], vals)
                return 0

            lax.fori_loop(0, NVEC - 1, move_vec, 0, unroll=8)
            sl = pl.ds((NVEC - 1) * LANES, LANES)
            vals = plsc.load_gather(rows_v.at[i], [gr_v[sl], gc_v[sl]])
            plsc.store_scatter(out_vs[i], [sp_v[sl]], vals, mask=tail_mask)

            pltpu.async_copy(out_vs[i], out_hbm.at[b], semo.at[i])

            @pl.when(g < NGRP - 1)
            def _():
                gather_cp(i, b + NBUF).start()
        return 0

    lax.fori_loop(0, NGRP, per_group, 0)

    for i in range(NBUF):
        pltpu.make_async_copy(out_hbm.at[0], out_vs[i], semo.at[i]).wait()


@jax.jit
def kernel(inputs):
    mesh = plsc.VectorSubcoreMesh(core_axis_name="c", subcore_axis_name="s")
    run = pl.kernel(
        _sc_body,
        out_type=jax.ShapeDtypeStruct((B, OUT_ROW), jnp.float32),
        mesh=mesh,
        compiler_params=pltpu.CompilerParams(use_tc_tiling_on_sc=True,
                                             needs_layout_passes=False),
        scratch_types=[
            pltpu.VMEM((NU_PAD,), jnp.int32),
            pltpu.VMEM((NVEC * LANES,), jnp.int32),
            pltpu.VMEM((NVEC * LANES,), jnp.int32),
            pltpu.VMEM((NVEC * LANES,), jnp.int32),
            pltpu.VMEM((NBUF, NU_PAD, N1), jnp.float32),
        ] + [pltpu.VMEM((OUT_ROW,), jnp.float32) for _ in range(NBUF)] + [
            pltpu.SemaphoreType.DMA((NBUF,)),
            pltpu.SemaphoreType.DMA((NBUF,)),
        ],
    )
    out = run(inputs, jnp.asarray(_u43), jnp.asarray(_gr), jnp.asarray(_gc),
              jnp.asarray(_sp))
    return out.reshape(B, 12, 12, NS_OUT)


# speedup vs baseline: 5.0355x; 1.0420x over previous
import jax
import jax.numpy as jnp
import numpy as np
from jax import lax
from jax.experimental import pallas as pl
from jax.experimental.pallas import tpu as pltpu
from jax.experimental.pallas import tpu_sc as plsc

N1 = 256
B = 1024
SIZES = tuple(range(5, 13))
NS_OUT = len(SIZES)
OUT_ROW = 12 * 12 * NS_OUT
LANES = 16

_scale_idx = {s: [i * (N1 - 1) // (s - 1) for i in range(s)] for s in SIZES}
_union = sorted({i for v in _scale_idx.values() for i in v})
NU = len(_union)
_pos = {v: i for i, v in enumerate(_union)}
NU_PAD = 48
_u43 = np.array(_union + [0] * (NU_PAD - NU), dtype=np.int32)

_gr, _gc, _sp = [], [], []
for r in range(12):
    for c in range(12):
        for si, s in enumerate(SIZES):
            if r < s and c < s:
                _gr.append(_pos[_scale_idx[s][r]])
                _gc.append(_scale_idx[s][c])
                _sp.append((r * 12 + c) * NS_OUT + si)
NVALID = len(_gr)
NVEC = -(-NVALID // LANES)
NTAIL = NVALID - (NVEC - 1) * LANES
_pad = NVEC * LANES - NVALID
_gr = np.array(_gr + [0] * _pad, dtype=np.int32)
_gc = np.array(_gc + [0] * _pad, dtype=np.int32)
_sp = np.array(_sp + [0] * _pad, dtype=np.int32)
_tab = np.concatenate([_u43, _gr, _gc, _sp])
_OFF_GR = NU_PAD
_OFF_GC = NU_PAD + NVEC * LANES
_OFF_SP = NU_PAD + 2 * NVEC * LANES

NC, NSUB = 2, 16
NW = NC * NSUB
B_PER_W = B // NW
NBUF = 8
NGRP = B_PER_W // NBUF


def _sc_body(in_hbm, tab_hbm, out_hbm, tab_v, rows_v, *rest):
    out_vs = rest[:NBUF]
    semg, semo = rest[NBUF], rest[NBUF + 1]
    wid = lax.axis_index("s") * NC + lax.axis_index("c")
    base = wid * B_PER_W

    pltpu.sync_copy(tab_hbm, tab_v)

    zeros = jnp.zeros((LANES,), jnp.float32)
    for i in range(NBUF):
        def zero_vec(j, _):
            out_vs[i][pl.ds(j * LANES, LANES)] = zeros
            return 0
        lax.fori_loop(0, OUT_ROW // LANES, zero_vec, 0)

    def gather_cp(i, b):
        return pltpu.make_async_copy(
            in_hbm.at[b].at[tab_v.at[pl.ds(0, NU_PAD)]],
            rows_v.at[i], semg.at[i])

    for i in range(NBUF):
        gather_cp(i, base + i).start()

    tail_mask = lax.iota(jnp.int32, LANES) < NTAIL

    def per_group(g, _):
        for i in range(NBUF):
            b = base + g * NBUF + i
            gather_cp(i, b).wait()

            @pl.when(g > 0)
            def _():
                pltpu.make_async_copy(out_hbm.at[0], out_vs[i],
                                      semo.at[i]).wait()

            def move_vec(k, _):
                o = k * LANES
                vals = plsc.load_gather(
                    rows_v.at[i], [tab_v[pl.ds(_OFF_GR + o, LANES)],
                                   tab_v[pl.ds(_OFF_GC + o, LANES)]])
                plsc.store_scatter(out_vs[i], [tab_v[pl.ds(_OFF_SP + o, LANES)]],
                                   vals)
                return 0

            lax.fori_loop(0, NVEC - 1, move_vec, 0, unroll=8)
            o = (NVEC - 1) * LANES
            vals = plsc.load_gather(
                rows_v.at[i], [tab_v[pl.ds(_OFF_GR + o, LANES)],
                               tab_v[pl.ds(_OFF_GC + o, LANES)]])
            plsc.store_scatter(out_vs[i], [tab_v[pl.ds(_OFF_SP + o, LANES)]],
                               vals, mask=tail_mask)

            pltpu.async_copy(out_vs[i], out_hbm.at[b], semo.at[i])

            @pl.when(g < NGRP - 1)
            def _():
                gather_cp(i, b + NBUF).start()
        return 0

    lax.fori_loop(0, NGRP, per_group, 0)

    for i in range(NBUF):
        pltpu.make_async_copy(out_hbm.at[0], out_vs[i], semo.at[i]).wait()


@jax.jit
def kernel(inputs):
    mesh = plsc.VectorSubcoreMesh(core_axis_name="c", subcore_axis_name="s")
    run = pl.kernel(
        _sc_body,
        out_type=jax.ShapeDtypeStruct((B, OUT_ROW), jnp.float32),
        mesh=mesh,
        compiler_params=pltpu.CompilerParams(use_tc_tiling_on_sc=True,
                                             needs_layout_passes=False),
        scratch_types=[
            pltpu.VMEM((_tab.size,), jnp.int32),
            pltpu.VMEM((NBUF, NU_PAD, N1), jnp.float32),
        ] + [pltpu.VMEM((OUT_ROW,), jnp.float32) for _ in range(NBUF)] + [
            pltpu.SemaphoreType.DMA((NBUF,)),
            pltpu.SemaphoreType.DMA((NBUF,)),
        ],
    )
    out = run(inputs, jnp.asarray(_tab))
    return out.reshape(B, 12, 12, NS_OUT)


# speedup vs baseline: 5.4886x; 1.0900x over previous
import jax
import jax.numpy as jnp
import numpy as np
from jax import lax
from jax.experimental import pallas as pl
from jax.experimental.pallas import tpu as pltpu
from jax.experimental.pallas import tpu_sc as plsc

N1 = 256
B = 1024
SIZES = tuple(range(5, 13))
NS_OUT = len(SIZES)
OUT_ROW = 12 * 12 * NS_OUT
LANES = 16

_scale_idx = {s: [i * (N1 - 1) // (s - 1) for i in range(s)] for s in SIZES}
_union = sorted({i for v in _scale_idx.values() for i in v})
NU = len(_union)
_pos = {v: i for i, v in enumerate(_union)}
NU_PAD = 48
_u43 = np.array(_union + [0] * (NU_PAD - NU), dtype=np.int32)

_gr, _gc, _sp = [], [], []
for r in range(12):
    for c in range(12):
        for si, s in enumerate(SIZES):
            if r < s and c < s:
                _gr.append(_pos[_scale_idx[s][r]])
                _gc.append(_scale_idx[s][c])
                _sp.append((r * 12 + c) * NS_OUT + si)
NVALID = len(_gr)
NVEC = -(-NVALID // LANES)
NTAIL = NVALID - (NVEC - 1) * LANES
_pad = NVEC * LANES - NVALID
_gr = np.array(_gr + [0] * _pad, dtype=np.int32)
_gc = np.array(_gc + [0] * _pad, dtype=np.int32)
_sp = np.array(_sp + [0] * _pad, dtype=np.int32)
_tab = np.concatenate([_u43, _gr, _gc, _sp])
_OFF_GR = NU_PAD
_OFF_GC = NU_PAD + NVEC * LANES
_OFF_SP = NU_PAD + 2 * NVEC * LANES

NC, NSUB = 2, 16
NW = NC * NSUB
B_PER_W = B // NW
NBUF = 8
NGRP = B_PER_W // NBUF


def _sc_body(in_hbm, tab_hbm, out_hbm, tab_v, rows_v, outb_v, semg, semo):
    wid = lax.axis_index("s") * NC + lax.axis_index("c")
    base = wid * B_PER_W

    pltpu.sync_copy(tab_hbm, tab_v)

    zeros = jnp.zeros((LANES,), jnp.float32)

    def zero_vec(j, _):
        outb_v[pl.ds(j * LANES, LANES)] = zeros
        return 0

    lax.fori_loop(0, NBUF * OUT_ROW // LANES, zero_vec, 0)

    def gather_cp(i, b):
        return pltpu.make_async_copy(
            in_hbm.at[b].at[tab_v.at[pl.ds(0, NU_PAD)]],
            rows_v.at[i], semg.at[i])

    def out_cp(i, b):
        return pltpu.make_async_copy(outb_v.at[pl.ds(i * OUT_ROW, OUT_ROW)],
                                     out_hbm.at[b], semo.at[i])

    def fire(i, _):
        gather_cp(i, base + i).start()
        return 0

    lax.fori_loop(0, NBUF, fire, 0)

    tail_mask = lax.iota(jnp.int32, LANES) < NTAIL

    def per_batch(bl, _):
        i = lax.rem(bl, NBUF)
        b = base + bl
        obase = i * OUT_ROW
        gather_cp(i, b).wait()

        @pl.when(bl >= NBUF)
        def _():
            out_cp(i, b).wait()

        def move_vec(k, _):
            o = k * LANES
            vals = plsc.load_gather(
                rows_v.at[i], [tab_v[pl.ds(_OFF_GR + o, LANES)],
                               tab_v[pl.ds(_OFF_GC + o, LANES)]])
            plsc.store_scatter(outb_v,
                               [tab_v[pl.ds(_OFF_SP + o, LANES)] + obase],
                               vals)
            return 0

        lax.fori_loop(0, NVEC - 1, move_vec, 0, unroll=8)
        o = (NVEC - 1) * LANES
        vals = plsc.load_gather(
            rows_v.at[i], [tab_v[pl.ds(_OFF_GR + o, LANES)],
                           tab_v[pl.ds(_OFF_GC + o, LANES)]])
        plsc.store_scatter(outb_v,
                           [tab_v[pl.ds(_OFF_SP + o, LANES)] + obase],
                           vals, mask=tail_mask)

        out_cp(i, b).start()

        @pl.when(bl < B_PER_W - NBUF)
        def _():
            gather_cp(i, b + NBUF).start()
        return 0

    lax.fori_loop(0, B_PER_W, per_batch, 0)

    def drain(i, _):
        out_cp(i, base).wait()
        return 0

    lax.fori_loop(0, NBUF, drain, 0)


@jax.jit
def kernel(inputs):
    mesh = plsc.VectorSubcoreMesh(core_axis_name="c", subcore_axis_name="s")
    run = pl.kernel(
        _sc_body,
        out_type=jax.ShapeDtypeStruct((B, OUT_ROW), jnp.float32),
        mesh=mesh,
        compiler_params=pltpu.CompilerParams(use_tc_tiling_on_sc=True,
                                             needs_layout_passes=False),
        scratch_types=[
            pltpu.VMEM((_tab.size,), jnp.int32),
            pltpu.VMEM((NBUF, NU_PAD, N1), jnp.float32),
            pltpu.VMEM((NBUF * OUT_ROW,), jnp.float32),
            pltpu.SemaphoreType.DMA((NBUF,)),
            pltpu.SemaphoreType.DMA((NBUF,)),
        ],
    )
    out = run(inputs, jnp.asarray(_tab))
    return out.reshape(B, 12, 12, NS_OUT)
